# baseline (speedup 1.0000x reference)
import functools

import jax
import jax.numpy as jnp
from jax import lax
from jax.experimental import pallas as pl
from jax.experimental.pallas import tpu as pltpu
from jax.experimental.pallas import tpu_sc as plsc

N = 20000
MAX_DET = 100
IOU_T = 0.5
L = 16
NB = N // L
NGRP = 79
NB_PAD = NGRP * L
S_PAD = NB_PAD * L
L1_PAD = 1280
NC2 = 5
NKC = 7
NEG = float("-inf")
BIG = jnp.int32(10_000)


def _lanes():
    return lax.iota(jnp.int32, 16)


def _vmax(v):
    return jnp.max(v)


def _ffs(mask):
    return plsc.all_reduce_ffs(mask)[0]


def _splat_i(x):
    return jnp.full((16,), x, jnp.int32)


def _splat_f(x):
    return jnp.full((16,), x, jnp.float32)


def _nms_body(boxes_hbm, scores_hbm, out_hbm,
              boxes_v, scores_v, l1_v, out_v, dma_sem):
    tile0 = (lax.axis_index("c") == 0) & (lax.axis_index("s") == 0)

    @pl.when(tile0)
    def _():
        lanes = _lanes()
        boxes_dma = pltpu.async_copy(boxes_hbm, boxes_v.at[pl.ds(0, N * 4)],
                                     dma_sem)
        pltpu.sync_copy(scores_hbm, scores_v.at[pl.ds(0, N)])
        for g in range(N // L, S_PAD // L):
            scores_v[pl.ds(g * L, L)] = _splat_f(NEG)
        l1_v[pl.ds(NB_PAD, L)] = _splat_f(NEG)
        for g in range(32):
            out_v[pl.ds(g * L, L)] = jnp.zeros((16,), jnp.float32)

        def build_l1(g, _):
            base = pl.multiple_of(g * 256, 256)
            acc = _splat_f(NEG)
            for j in range(16):
                m_j = _vmax(scores_v[pl.ds(base + j * L, L)])
                acc = jnp.where(lanes == j, m_j, acc)
            l1_v[pl.ds(pl.multiple_of(g * L, L), L)] = acc
            return 0

        lax.fori_loop(0, NGRP, build_l1, 0)

        c_init = []
        for c in range(NC2):
            acc = _splat_f(NEG)
            for j in range(16):
                m_j = _vmax(l1_v[pl.ds((c * 16 + j) * L, L)])
                acc = jnp.where(lanes == j, m_j, acc)
            c_init.append(acc)
        m0 = _vmax(jnp.maximum(
            jnp.maximum(jnp.maximum(c_init[0], c_init[1]),
                        jnp.maximum(c_init[2], c_init[3])), c_init[4]))

        boxes_dma.wait()

        zf = jnp.zeros((16,), jnp.float32)
        k_init = tuple(zf for _ in range(5 * NKC))

        def cond(carry):
            kept, done = carry[0], carry[1]
            return (kept < MAX_DET) & (done == 0)

        def body(carry):
            kept, done, m, l2, ks = carry
            valid = m > NEG

            j = BIG
            for c in range(NC2):
                lane_c = _ffs(l2[c] == m)
                j = jnp.minimum(
                    j, jnp.where(lane_c < 16, c * 16 + lane_c, BIG))
            j = jnp.minimum(j, jnp.int32(79))
            cj = j >> 4
            lane_j = j & 15
            l1c = l1_v[pl.ds(pl.multiple_of(j * L, L), L)]
            lane_b = jnp.minimum(_ffs(l1c == m), 15)
            b = j * 16 + lane_b
            sc = scores_v[pl.ds(pl.multiple_of(b * L, L), L)]
            lane_i = jnp.minimum(_ffs(sc == m), 15)
            idx = jnp.minimum(b * 16 + lane_i, jnp.int32(N - 1))

            g = boxes_v[pl.ds(pl.multiple_of(idx * 4, 4), L)]
            bx1 = g[0]
            by1 = g[1]
            bx2 = g[2]
            by2 = g[3]
            barea = (bx2 - bx1) * (by2 - by1)

            kx1 = ks[0:NKC]
            ky1 = ks[NKC:2 * NKC]
            kx2 = ks[2 * NKC:3 * NKC]
            ky2 = ks[3 * NKC:4 * NKC]
            kar = ks[4 * NKC:5 * NKC]
            sup = jnp.zeros((16,), jnp.bool_)
            for k in range(NKC):
                live = (k * 16 + lanes) < kept
                xx1 = jnp.maximum(bx1, kx1[k])
                yy1 = jnp.maximum(by1, ky1[k])
                xx2 = jnp.minimum(bx2, kx2[k])
                yy2 = jnp.minimum(by2, ky2[k])
                inter = (jnp.maximum(xx2 - xx1, 0.0)
                         * jnp.maximum(yy2 - yy1, 0.0))
                union = barea + kar[k] - inter
                iou = inter / jnp.maximum(union, 1e-9)
                sup = sup | (live & (iou > IOU_T))
            keep = valid & (plsc.all_reduce_population_count(sup)[0] == 0)

            kc = kept >> 4
            kl = kept & 15
            new_ks = []
            for vals, arr in ((bx1, kx1), (by1, ky1), (bx2, kx2),
                              (by2, ky2), (barea, kar)):
                for k in range(NKC):
                    hit = keep & (kc == k) & (lanes == kl)
                    new_ks.append(jnp.where(hit, vals, arr[k]))
            row = jnp.where(lanes == 0, bx1,
                  jnp.where(lanes == 1, by1,
                  jnp.where(lanes == 2, bx2,
                  jnp.where(lanes == 3, by2, m))))
            plsc.store_scatter(out_v, [kept * 5 + lanes], row,
                               mask=(lanes < 5) & keep)

            vmask = (lanes == 0) & valid
            nm1 = _vmax(jnp.where(lanes == lane_i, _splat_f(NEG), sc))
            nm2x = _vmax(jnp.where(lanes == lane_b, _splat_f(NEG), l1c))
            plsc.store_scatter(scores_v, [_splat_i(idx)], _splat_f(NEG),
                               mask=vmask)
            nm2 = jnp.maximum(nm2x, nm1)
            plsc.store_scatter(l1_v, [_splat_i(b)], _splat_f(nm1), mask=vmask)
            new_l2 = []
            mx = _splat_f(NEG)
            for c in range(NC2):
                upd = valid & (cj == c)
                l2c = jnp.where(upd & (lanes == lane_j), nm2, l2[c])
                new_l2.append(l2c)
                mx = jnp.maximum(mx, l2c)
            m_new = jnp.where(valid, _vmax(mx), m)

            kept = kept + jnp.where(keep, 1, 0).astype(jnp.int32)
            done = jnp.where(valid, 0, 1).astype(jnp.int32)
            return kept, done, m_new, tuple(new_l2), tuple(new_ks)

        lax.while_loop(cond, body,
                       (jnp.int32(0), jnp.int32(0), m0,
                        tuple(c_init), k_init))
        pltpu.sync_copy(out_v, out_hbm)


@jax.jit
def kernel(boxes, scores):
    f = functools.partial(
        pl.kernel,
        mesh=plsc.VectorSubcoreMesh(core_axis_name="c", subcore_axis_name="s",
                                    num_cores=1),
        compiler_params=pltpu.CompilerParams(needs_layout_passes=False,
                                             skip_device_barrier=True),
        out_type=jax.ShapeDtypeStruct((512,), jnp.float32),
        scratch_types=[
            pltpu.VMEM((N * 4 + L,), jnp.float32),
            pltpu.VMEM((S_PAD,), jnp.float32),
            pltpu.VMEM((L1_PAD,), jnp.float32),
            pltpu.VMEM((512,), jnp.float32),
            pltpu.SemaphoreType.DMA,
        ],
    )(_nms_body)
    out = f(boxes.reshape(N * 4), scores)
    return out[: MAX_DET * 5].reshape(MAX_DET, 5)


# speedup vs baseline: 1.0338x; 1.0338x over previous
import functools

import jax
import jax.numpy as jnp
from jax import lax
from jax.experimental import pallas as pl
from jax.experimental.pallas import tpu as pltpu
from jax.experimental.pallas import tpu_sc as plsc

N = 20000
MAX_DET = 100
IOU_T = 0.5
L = 16
NB = N // L
NGRP = 79
NB_PAD = NGRP * L
S_PAD = NB_PAD * L
L1_PAD = 1280
NC2 = 5
NKC = 7
NEG = float("-inf")
BIG = 10_000


def _lanes():
    return lax.iota(jnp.int32, 16)


def _vmax(v):
    return jnp.max(v)


def _ffs(mask):
    return plsc.all_reduce_ffs(mask)[0]


def _splat_i(x):
    return jnp.full((16,), x, jnp.int32)


def _splat_f(x):
    return jnp.full((16,), x, jnp.float32)


def _nms_body(boxes_hbm, scores_hbm, out_hbm,
              boxes_v, scores_v, l1_v,
              kx1_v, ky1_v, kx2_v, ky2_v, kar_v, out_v, dma_sem):
    tile0 = (lax.axis_index("c") == 0) & (lax.axis_index("s") == 0)

    @pl.when(tile0)
    def _():
        lanes = _lanes()
        boxes_dma = pltpu.async_copy(boxes_hbm, boxes_v.at[pl.ds(0, N * 4)],
                                     dma_sem)
        pltpu.sync_copy(scores_hbm, scores_v.at[pl.ds(0, N)])
        for g in range(N // L, S_PAD // L):
            scores_v[pl.ds(g * L, L)] = _splat_f(NEG)
        l1_v[pl.ds(NB_PAD, L)] = _splat_f(NEG)
        for g in range(32):
            out_v[pl.ds(g * L, L)] = jnp.zeros((16,), jnp.float32)

        def build_l1(g, _):
            base = pl.multiple_of(g * 256, 256)
            acc = _splat_f(NEG)
            for j in range(16):
                m_j = _vmax(scores_v[pl.ds(base + j * L, L)])
                acc = jnp.where(lanes == j, m_j, acc)
            l1_v[pl.ds(pl.multiple_of(g * L, L), L)] = acc
            return 0

        lax.fori_loop(0, NGRP, build_l1, 0)

        c_init = []
        for c in range(NC2):
            acc = _splat_f(NEG)
            for j in range(16):
                m_j = _vmax(l1_v[pl.ds((c * 16 + j) * L, L)])
                acc = jnp.where(lanes == j, m_j, acc)
            c_init.append(acc)
        m0 = _vmax(jnp.maximum(
            jnp.maximum(jnp.maximum(c_init[0], c_init[1]),
                        jnp.maximum(c_init[2], c_init[3])), c_init[4]))

        boxes_dma.wait()


        def cond(carry):
            kept, done = carry[0], carry[1]
            return (kept < MAX_DET) & (done == 0)

        def body(carry):
            kept, done, m, l2 = carry
            valid = m > NEG

            j = jnp.int32(BIG)
            for c in range(NC2):
                lane_c = _ffs(l2[c] == m)
                j = jnp.minimum(
                    j, jnp.where(lane_c < 16, c * 16 + lane_c, BIG))
            j = jnp.minimum(j, jnp.int32(79))
            cj = j >> 4
            lane_j = j & 15
            l1c = l1_v[pl.ds(pl.multiple_of(j * L, L), L)]
            lane_b = jnp.minimum(_ffs(l1c == m), 15)
            b = j * 16 + lane_b
            sc = scores_v[pl.ds(pl.multiple_of(b * L, L), L)]
            lane_i = jnp.minimum(_ffs(sc == m), 15)
            idx = jnp.minimum(b * 16 + lane_i, jnp.int32(N - 1))

            g = boxes_v[pl.ds(pl.multiple_of(idx * 4, 4), L)]
            bx1 = g[0]
            by1 = g[1]
            bx2 = g[2]
            by2 = g[3]
            barea = (bx2 - bx1) * (by2 - by1)

            sup = jnp.zeros((16,), jnp.bool_)
            for k in range(NKC):
                live = (k * 16 + lanes) < kept
                xx1 = jnp.maximum(bx1, kx1_v[pl.ds(k * L, L)])
                yy1 = jnp.maximum(by1, ky1_v[pl.ds(k * L, L)])
                xx2 = jnp.minimum(bx2, kx2_v[pl.ds(k * L, L)])
                yy2 = jnp.minimum(by2, ky2_v[pl.ds(k * L, L)])
                inter = (jnp.maximum(xx2 - xx1, 0.0)
                         * jnp.maximum(yy2 - yy1, 0.0))
                union = barea + kar_v[pl.ds(k * L, L)] - inter
                iou = inter / jnp.maximum(union, 1e-9)
                sup = sup | (live & (iou > IOU_T))
            keep = valid & (plsc.all_reduce_population_count(sup)[0] == 0)

            app = (lanes == 0) & keep
            kidx = _splat_i(kept)
            plsc.store_scatter(kx1_v, [kidx], _splat_f(bx1), mask=app)
            plsc.store_scatter(ky1_v, [kidx], _splat_f(by1), mask=app)
            plsc.store_scatter(kx2_v, [kidx], _splat_f(bx2), mask=app)
            plsc.store_scatter(ky2_v, [kidx], _splat_f(by2), mask=app)
            plsc.store_scatter(kar_v, [kidx], _splat_f(barea), mask=app)
            row = jnp.where(lanes == 0, bx1,
                  jnp.where(lanes == 1, by1,
                  jnp.where(lanes == 2, bx2,
                  jnp.where(lanes == 3, by2, m))))
            plsc.store_scatter(out_v, [kept * 5 + lanes], row,
                               mask=(lanes < 5) & keep)

            vmask = (lanes == 0) & valid
            nm1 = _vmax(jnp.where(lanes == lane_i, _splat_f(NEG), sc))
            nm2x = _vmax(jnp.where(lanes == lane_b, _splat_f(NEG), l1c))
            plsc.store_scatter(scores_v, [_splat_i(idx)], _splat_f(NEG),
                               mask=vmask)
            nm2 = jnp.maximum(nm2x, nm1)
            plsc.store_scatter(l1_v, [_splat_i(b)], _splat_f(nm1), mask=vmask)
            new_l2 = []
            mx = _splat_f(NEG)
            for c in range(NC2):
                upd = valid & (cj == c)
                l2c = jnp.where(upd & (lanes == lane_j), nm2, l2[c])
                new_l2.append(l2c)
                mx = jnp.maximum(mx, l2c)
            m_new = jnp.where(valid, _vmax(mx), m)

            kept = kept + jnp.where(keep, 1, 0).astype(jnp.int32)
            done = jnp.where(valid, 0, 1).astype(jnp.int32)
            return kept, done, m_new, tuple(new_l2)

        lax.while_loop(cond, body,
                       (jnp.int32(0), jnp.int32(0), m0, tuple(c_init)))
        pltpu.sync_copy(out_v, out_hbm)


@jax.jit
def kernel(boxes, scores):
    f = functools.partial(
        pl.kernel,
        mesh=plsc.VectorSubcoreMesh(core_axis_name="c", subcore_axis_name="s",
                                    num_cores=1),
        compiler_params=pltpu.CompilerParams(needs_layout_passes=False,
                                             skip_device_barrier=True),
        out_type=jax.ShapeDtypeStruct((512,), jnp.float32),
        scratch_types=[
            pltpu.VMEM((N * 4 + L,), jnp.float32),
            pltpu.VMEM((S_PAD,), jnp.float32),
            pltpu.VMEM((L1_PAD,), jnp.float32),
            pltpu.VMEM((NKC * L,), jnp.float32),
            pltpu.VMEM((NKC * L,), jnp.float32),
            pltpu.VMEM((NKC * L,), jnp.float32),
            pltpu.VMEM((NKC * L,), jnp.float32),
            pltpu.VMEM((NKC * L,), jnp.float32),
            pltpu.VMEM((512,), jnp.float32),
            pltpu.SemaphoreType.DMA,
        ],
    )(_nms_body)
    out = f(boxes.reshape(N * 4), scores)
    return out[: MAX_DET * 5].reshape(MAX_DET, 5)
